# baseline (speedup 1.0000x reference)
import functools

import jax
import jax.numpy as jnp
from jax import lax
from jax.experimental import pallas as pl
from jax.experimental.pallas import tpu as pltpu
from jax.experimental.pallas import tpu_sc as plsc

NC = 2
NS = 16
NW = NC * NS

BBLK = 512
TPC = BBLK // 128


def _emb_body(hist, d, table_hbm, idx_hbm, out_hbm,
              idx_v, g0, g1, r0, r1, semg0, semg1, semw0, semw1):
    c = lax.axis_index("c")
    s = lax.axis_index("s")
    wid = s * NC + c
    ct = d // 8

    pltpu.sync_copy(idx_hbm.at[:, pl.ds(wid * BBLK, BBLK)], idx_v)

    bufs = (g0, g1)
    rbufs = (r0, r1)
    semg = (semg0, semg1)
    semw = (semw0, semw1)

    def start_gather(h, slot):
        pltpu.async_copy(table_hbm.at[idx_v.at[h]],
                         bufs[slot], semg[slot])

    def wait_gather(h, slot):
        pltpu.make_async_copy(table_hbm.at[idx_v.at[h]],
                              bufs[slot], semg[slot]).wait()

    def start_write(h, slot):
        for i in range(ct):
            pltpu.async_copy(rbufs[slot].at[i],
                             out_hbm.at[h * ct + i, pl.ds(wid * TPC, TPC)],
                             semw[slot])

    def wait_write(h, slot):
        for i in range(ct):
            pltpu.make_async_copy(rbufs[slot].at[i],
                                  out_hbm.at[h * ct + i, pl.ds(wid * TPC, TPC)],
                                  semw[slot]).wait()

    iota = lax.iota(jnp.int32, 16)
    cols = [jnp.full((16,), cc, jnp.int32) for cc in range(d)]

    def transpose_chunk(slot):
        g = bufs[slot]
        r = rbufs[slot]

        @plsc.parallel_loop(0, TPC * 8, unroll=4)
        def _(tb):
            tl = tb // 8
            blk = tb % 8
            rows = tl * 128 + blk * 16 + iota
            for i in range(ct):
                for cl in range(8):
                    r[i, tl, cl, pl.ds(blk * 16, 16)] = plsc.load_gather(
                        g, [rows, cols[i * 8 + cl]])

    start_gather(0, 0)
    start_gather(1, 1)

    @pl.loop(0, hist // 2)
    def _(p):
        h = p * 2
        for slot in (0, 1):
            hh = h + slot
            wait_gather(hh, slot)

            @pl.when(p > 0)
            def _():
                wait_write(hh - 2, slot)

            transpose_chunk(slot)
            start_write(hh, slot)

            @pl.when(hh + 2 < hist)
            def _():
                start_gather(hh + 2, slot)

    wait_write(hist - 2, 0)
    wait_write(hist - 1, 1)


@functools.partial(jax.jit, static_argnames=("b", "hist", "d"))
def _sc_emb(table, idx_t, b, hist, d):
    assert b % (NW * BBLK) == 0 and hist % 2 == 0 and d % 8 == 0
    ct = d // 8
    mesh = plsc.VectorSubcoreMesh(
        core_axis_name="c", subcore_axis_name="s", num_cores=NC, num_subcores=NS
    )
    ltile = pl.kernel(
        functools.partial(_emb_body, hist, d),
        out_type=jax.ShapeDtypeStruct((hist * ct, b // 128, 8, 128),
                                      table.dtype),
        mesh=mesh,
        scratch_types=[
            pltpu.VMEM((hist, BBLK), jnp.int32),
            pltpu.VMEM((BBLK, d), table.dtype),
            pltpu.VMEM((BBLK, d), table.dtype),
            pltpu.VMEM((ct, TPC, 8, 128), table.dtype),
            pltpu.VMEM((ct, TPC, 8, 128), table.dtype),
            pltpu.SemaphoreType.DMA,
            pltpu.SemaphoreType.DMA,
            pltpu.SemaphoreType.DMA,
            pltpu.SemaphoreType.DMA,
        ],
        compiler_params=pltpu.CompilerParams(
            use_tc_tiling_on_sc=False, needs_layout_passes=False),
    )(table, idx_t)
    l5 = ltile.reshape(hist, ct, b // 128, 8, 128)
    return jnp.transpose(l5, (2, 4, 0, 1, 3)).reshape(b, hist, d)


def kernel(indices, table):
    b, h = indices.shape
    d = table.shape[1]
    idx_t = jnp.transpose(indices).astype(jnp.int32)
    return _sc_emb(table, idx_t, b, h, d)


# speedup vs baseline: 1.1428x; 1.1428x over previous
import functools

import jax
import jax.numpy as jnp
from jax import lax
from jax.experimental import pallas as pl
from jax.experimental.pallas import tpu as pltpu
from jax.experimental.pallas import tpu_sc as plsc

NC = 2
NS = 16
NW = NC * NS

BBLK = 512
TPC = BBLK // 128


def _emb_body(hist, d, table_hbm, idx_hbm, out_hbm,
              idx_v, g0, g1, r0, r1, semg0, semg1, semw0, semw1):
    c = lax.axis_index("c")
    s = lax.axis_index("s")
    wid = s * NC + c
    ct = d // 8

    pltpu.sync_copy(idx_hbm.at[:, pl.ds(wid * BBLK, BBLK)], idx_v)

    bufs = (g0, g1)
    rbufs = (r0, r1)
    semg = (semg0, semg1)
    semw = (semw0, semw1)

    def start_gather(h, slot):
        pltpu.async_copy(table_hbm.at[idx_v.at[h]],
                         bufs[slot], semg[slot])

    def wait_gather(h, slot):
        pltpu.make_async_copy(table_hbm.at[idx_v.at[h]],
                              bufs[slot], semg[slot]).wait()

    def start_write(h, slot):
        for i in range(ct):
            pltpu.async_copy(rbufs[slot].at[i],
                             out_hbm.at[h * ct + i, pl.ds(wid * TPC, TPC)],
                             semw[slot])

    def wait_write(h, slot):
        for i in range(ct):
            pltpu.make_async_copy(rbufs[slot].at[i],
                                  out_hbm.at[h * ct + i, pl.ds(wid * TPC, TPC)],
                                  semw[slot]).wait()

    iota = lax.iota(jnp.int32, 16)
    cols = [jnp.full((16,), cc, jnp.int32) for cc in range(d)]

    def transpose_chunk(slot):
        g = bufs[slot]
        r = rbufs[slot]

        @plsc.parallel_loop(0, TPC * 8, unroll=2)
        def _(tb):
            tl = tb // 8
            blk = tb % 8
            rows = tl * 128 + blk * 16 + iota
            for i in range(ct):
                for cl in range(8):
                    r[i, tl, cl, pl.ds(blk * 16, 16)] = plsc.load_gather(
                        g, [rows, cols[i * 8 + cl]])

    start_gather(0, 0)
    start_gather(1, 1)

    @pl.loop(0, hist // 2)
    def _(p):
        h = p * 2
        for slot in (0, 1):
            hh = h + slot
            wait_gather(hh, slot)

            @pl.when(p > 0)
            def _():
                wait_write(hh - 2, slot)

            transpose_chunk(slot)
            start_write(hh, slot)

            @pl.when(hh + 2 < hist)
            def _():
                start_gather(hh + 2, slot)

    wait_write(hist - 2, 0)
    wait_write(hist - 1, 1)


def _detile_body(v, d, table_t_hbm, tail_hbm, out_hbm,
                 t0, t1, s0, s1, semi0, semi1, semo0, semo1):
    c = lax.axis_index("c")
    s = lax.axis_index("s")
    wid = s * NC + c
    ct = d // 8
    nfull = v // 128
    rem = v - nfull * 128
    per = nfull // NW
    extra = nfull - per * NW
    base = per * wid + jnp.minimum(wid, extra)
    cnt = per + jnp.where(wid < extra, 1, 0)

    tb = (t0, t1)
    sb = (s0, s1)
    semi = (semi0, semi1)
    semo = (semo0, semo1)
    orow = d

    def load(j, slot):
        for i in range(ct):
            pltpu.async_copy(
                table_t_hbm.at[pl.ds(8 * i, 8), pl.ds(j * 128, 128)],
                tb[slot].at[pl.ds(8 * i, 8)], semi[slot])

    def wait_load(j, slot):
        for i in range(ct):
            pltpu.make_async_copy(
                table_t_hbm.at[pl.ds(8 * i, 8), pl.ds(j * 128, 128)],
                tb[slot].at[pl.ds(8 * i, 8)], semi[slot]).wait()

    def store(j, slot):
        pltpu.async_copy(sb[slot], out_hbm.at[pl.ds(j * orow, orow)],
                         semo[slot])

    def wait_store(j, slot):
        pltpu.make_async_copy(sb[slot], out_hbm.at[pl.ds(j * orow, orow)],
                              semo[slot]).wait()

    iota = lax.iota(jnp.int32, 16)
    rows0 = iota
    rows1 = iota + 16

    def process(slot):
        t = tb[slot]
        sbuf = sb[slot]

        @plsc.parallel_loop(0, orow, unroll=2)
        def _(u):
            for q in range(8):
                rl = u * 4 + q // 2
                rows = rows1 if (q % 2) else rows0
                sbuf[u, pl.ds(q * 16, 16)] = plsc.load_gather(
                    t, [rows, jnp.full((16,), 0, jnp.int32) + rl])

    pairs = cnt // 2

    @pl.when(cnt > 0)
    def _():
        load(base, 0)

    @pl.when(cnt > 1)
    def _():
        load(base + 1, 1)

    @pl.loop(0, pairs)
    def _(p):
        j = base + 2 * p
        for slot in (0, 1):
            jj = j + slot
            wait_load(jj, slot)

            @pl.when(p > 0)
            def _():
                wait_store(jj - 2, slot)

            process(slot)
            store(jj, slot)

            @pl.when(2 * p + 2 + slot < cnt)
            def _():
                load(jj + 2, slot)

    @pl.when(cnt % 2 == 1)
    def _():
        j = base + cnt - 1
        wait_load(j, 0)

        @pl.when(cnt > 2)
        def _():
            wait_store(j - 2, 0)

        process(0)
        store(j, 0)

    even = cnt % 2 == 0

    @pl.when(even)
    def _():
        wait_store(base + cnt - 2, 0)
        wait_store(base + cnt - 1, 1)

    @pl.when(jnp.logical_not(even))
    def _():
        @pl.when(cnt > 1)
        def _():
            wait_store(base + cnt - 2, 1)

        wait_store(base + cnt - 1, 0)

    if rem:
        prow = rem * d // 128

        @pl.when(wid == NW - 1)
        def _():
            pltpu.sync_copy(tail_hbm, s0.at[pl.ds(0, prow)])
            pltpu.sync_copy(s0.at[pl.ds(0, prow)],
                            out_hbm.at[pl.ds(nfull * orow, prow)])


@functools.partial(jax.jit, static_argnames=("v", "d"))
def _sc_detile(table_t, tail, v, d):
    assert d % 8 == 0 and (v * d) % 128 == 0
    mesh = plsc.VectorSubcoreMesh(
        core_axis_name="c", subcore_axis_name="s", num_cores=NC, num_subcores=NS
    )
    return pl.kernel(
        functools.partial(_detile_body, v, d),
        out_type=jax.ShapeDtypeStruct((v * d // 128, 128), table_t.dtype),
        mesh=mesh,
        scratch_types=[
            pltpu.VMEM((d, 128), table_t.dtype),
            pltpu.VMEM((d, 128), table_t.dtype),
            pltpu.VMEM((d, 128), table_t.dtype),
            pltpu.VMEM((d, 128), table_t.dtype),
            pltpu.SemaphoreType.DMA,
            pltpu.SemaphoreType.DMA,
            pltpu.SemaphoreType.DMA,
            pltpu.SemaphoreType.DMA,
        ],
        compiler_params=pltpu.CompilerParams(
            use_tc_tiling_on_sc=True, needs_layout_passes=False),
    )(table_t, tail)


@functools.partial(jax.jit, static_argnames=("b", "hist", "d"))
def _sc_emb(table, idx_t, b, hist, d):
    assert b % (NW * BBLK) == 0 and hist % 2 == 0 and d % 8 == 0
    ct = d // 8
    mesh = plsc.VectorSubcoreMesh(
        core_axis_name="c", subcore_axis_name="s", num_cores=NC, num_subcores=NS
    )
    ltile = pl.kernel(
        functools.partial(_emb_body, hist, d),
        out_type=jax.ShapeDtypeStruct((hist * ct, b // 128, 8, 128),
                                      table.dtype),
        mesh=mesh,
        scratch_types=[
            pltpu.VMEM((hist, BBLK), jnp.int32),
            pltpu.VMEM((BBLK, d), table.dtype),
            pltpu.VMEM((BBLK, d), table.dtype),
            pltpu.VMEM((ct, TPC, 8, 128), table.dtype),
            pltpu.VMEM((ct, TPC, 8, 128), table.dtype),
            pltpu.SemaphoreType.DMA,
            pltpu.SemaphoreType.DMA,
            pltpu.SemaphoreType.DMA,
            pltpu.SemaphoreType.DMA,
        ],
        compiler_params=pltpu.CompilerParams(
            use_tc_tiling_on_sc=False, needs_layout_passes=False),
    )(table, idx_t)
    l5 = ltile.reshape(hist, ct, b // 128, 8, 128)
    return jnp.transpose(l5, (2, 4, 0, 1, 3)).reshape(b, hist, d)


def kernel(indices, table):
    b, h = indices.shape
    v, d = table.shape
    idx_t = jnp.transpose(indices).astype(jnp.int32)
    nfull = v // 128
    rem = v - nfull * 128
    if rem:
        tail = table[nfull * 128:].reshape(rem * d // 128, 128)
    else:
        tail = jnp.zeros((1, 128), table.dtype)
    table_lin = _sc_detile(jnp.transpose(table), tail, v, d).reshape(v, d)
    return _sc_emb(table_lin, idx_t, b, h, d)


# speedup vs baseline: 1.1432x; 1.0003x over previous
import functools

import jax
import jax.numpy as jnp
from jax import lax
from jax.experimental import pallas as pl
from jax.experimental.pallas import tpu as pltpu
from jax.experimental.pallas import tpu_sc as plsc

NC = 2
NS = 16
NW = NC * NS

BBLK = 512
TPC = BBLK // 128


def _emb_body(hist, d, table_hbm, idx_hbm, out_hbm,
              idx_v, g0, g1, r0, r1, semg0, semg1, semw0, semw1):
    c = lax.axis_index("c")
    s = lax.axis_index("s")
    wid = s * NC + c
    ct = d // 8

    pltpu.sync_copy(idx_hbm.at[:, pl.ds(wid * BBLK, BBLK)], idx_v)

    bufs = (g0, g1)
    rbufs = (r0, r1)
    semg = (semg0, semg1)
    semw = (semw0, semw1)

    def start_gather(h, slot):
        pltpu.async_copy(table_hbm.at[idx_v.at[h]],
                         bufs[slot], semg[slot])

    def wait_gather(h, slot):
        pltpu.make_async_copy(table_hbm.at[idx_v.at[h]],
                              bufs[slot], semg[slot]).wait()

    def start_write(h, slot):
        for i in range(ct):
            pltpu.async_copy(rbufs[slot].at[i],
                             out_hbm.at[h * ct + i, pl.ds(wid * TPC, TPC)],
                             semw[slot])

    def wait_write(h, slot):
        for i in range(ct):
            pltpu.make_async_copy(rbufs[slot].at[i],
                                  out_hbm.at[h * ct + i, pl.ds(wid * TPC, TPC)],
                                  semw[slot]).wait()

    iota = lax.iota(jnp.int32, 16)
    cols = [jnp.full((16,), cc, jnp.int32) for cc in range(d)]

    def transpose_chunk(slot):
        g = bufs[slot]
        r = rbufs[slot]

        @plsc.parallel_loop(0, TPC * 8, unroll=2)
        def _(tb):
            tl = tb // 8
            blk = tb % 8
            rows = tl * 128 + blk * 16 + iota
            for i in range(ct):
                for cl in range(8):
                    r[i, tl, cl, pl.ds(blk * 16, 16)] = plsc.load_gather(
                        g, [rows, cols[i * 8 + cl]])

    start_gather(0, 0)
    start_gather(1, 1)

    @pl.loop(0, hist // 2)
    def _(p):
        h = p * 2
        for slot in (0, 1):
            hh = h + slot
            wait_gather(hh, slot)

            @pl.when(p > 0)
            def _():
                wait_write(hh - 2, slot)

            transpose_chunk(slot)
            start_write(hh, slot)

            @pl.when(hh + 2 < hist)
            def _():
                start_gather(hh + 2, slot)

    wait_write(hist - 2, 0)
    wait_write(hist - 1, 1)


def _detile_body(v, d, table_t_hbm, tail_hbm, out_hbm,
                 t0, t1, s0, s1, semi0, semi1, semo0, semo1):
    c = lax.axis_index("c")
    s = lax.axis_index("s")
    wid = s * NC + c
    ct = d // 8
    nfull = v // 128
    rem = v - nfull * 128
    per = nfull // NW
    extra = nfull - per * NW
    base = per * wid + jnp.minimum(wid, extra)
    cnt = per + jnp.where(wid < extra, 1, 0)

    tb = (t0, t1)
    sb = (s0, s1)
    semi = (semi0, semi1)
    semo = (semo0, semo1)
    orow = d

    def load(j, slot):
        for i in range(ct):
            pltpu.async_copy(
                table_t_hbm.at[pl.ds(8 * i, 8), pl.ds(j * 128, 128)],
                tb[slot].at[pl.ds(8 * i, 8)], semi[slot])

    def wait_load(j, slot):
        for i in range(ct):
            pltpu.make_async_copy(
                table_t_hbm.at[pl.ds(8 * i, 8), pl.ds(j * 128, 128)],
                tb[slot].at[pl.ds(8 * i, 8)], semi[slot]).wait()

    def store(j, slot):
        pltpu.async_copy(sb[slot], out_hbm.at[pl.ds(j * orow, orow)],
                         semo[slot])

    def wait_store(j, slot):
        pltpu.make_async_copy(sb[slot], out_hbm.at[pl.ds(j * orow, orow)],
                              semo[slot]).wait()

    iota = lax.iota(jnp.int32, 16)
    rows0 = iota
    rows1 = iota + 16

    def process(slot):
        t = tb[slot]
        sbuf = sb[slot]

        @plsc.parallel_loop(0, orow, unroll=4)
        def _(u):
            for q in range(8):
                rl = u * 4 + q // 2
                rows = rows1 if (q % 2) else rows0
                sbuf[u, pl.ds(q * 16, 16)] = plsc.load_gather(
                    t, [rows, jnp.full((16,), 0, jnp.int32) + rl])

    pairs = cnt // 2

    @pl.when(cnt > 0)
    def _():
        load(base, 0)

    @pl.when(cnt > 1)
    def _():
        load(base + 1, 1)

    @pl.loop(0, pairs)
    def _(p):
        j = base + 2 * p
        for slot in (0, 1):
            jj = j + slot
            wait_load(jj, slot)

            @pl.when(p > 0)
            def _():
                wait_store(jj - 2, slot)

            process(slot)
            store(jj, slot)

            @pl.when(2 * p + 2 + slot < cnt)
            def _():
                load(jj + 2, slot)

    @pl.when(cnt % 2 == 1)
    def _():
        j = base + cnt - 1
        wait_load(j, 0)

        @pl.when(cnt > 2)
        def _():
            wait_store(j - 2, 0)

        process(0)
        store(j, 0)

    even = cnt % 2 == 0

    @pl.when(even)
    def _():
        wait_store(base + cnt - 2, 0)
        wait_store(base + cnt - 1, 1)

    @pl.when(jnp.logical_not(even))
    def _():
        @pl.when(cnt > 1)
        def _():
            wait_store(base + cnt - 2, 1)

        wait_store(base + cnt - 1, 0)

    if rem:
        prow = rem * d // 128

        @pl.when(wid == NW - 1)
        def _():
            pltpu.sync_copy(tail_hbm, s0.at[pl.ds(0, prow)])
            pltpu.sync_copy(s0.at[pl.ds(0, prow)],
                            out_hbm.at[pl.ds(nfull * orow, prow)])


@functools.partial(jax.jit, static_argnames=("v", "d"))
def _sc_detile(table_t, tail, v, d):
    assert d % 8 == 0 and (v * d) % 128 == 0
    mesh = plsc.VectorSubcoreMesh(
        core_axis_name="c", subcore_axis_name="s", num_cores=NC, num_subcores=NS
    )
    return pl.kernel(
        functools.partial(_detile_body, v, d),
        out_type=jax.ShapeDtypeStruct((v * d // 128, 128), table_t.dtype),
        mesh=mesh,
        scratch_types=[
            pltpu.VMEM((d, 128), table_t.dtype),
            pltpu.VMEM((d, 128), table_t.dtype),
            pltpu.VMEM((d, 128), table_t.dtype),
            pltpu.VMEM((d, 128), table_t.dtype),
            pltpu.SemaphoreType.DMA,
            pltpu.SemaphoreType.DMA,
            pltpu.SemaphoreType.DMA,
            pltpu.SemaphoreType.DMA,
        ],
        compiler_params=pltpu.CompilerParams(
            use_tc_tiling_on_sc=True, needs_layout_passes=False),
    )(table_t, tail)


@functools.partial(jax.jit, static_argnames=("b", "hist", "d"))
def _sc_emb(table, idx_t, b, hist, d):
    assert b % (NW * BBLK) == 0 and hist % 2 == 0 and d % 8 == 0
    ct = d // 8
    mesh = plsc.VectorSubcoreMesh(
        core_axis_name="c", subcore_axis_name="s", num_cores=NC, num_subcores=NS
    )
    ltile = pl.kernel(
        functools.partial(_emb_body, hist, d),
        out_type=jax.ShapeDtypeStruct((hist * ct, b // 128, 8, 128),
                                      table.dtype),
        mesh=mesh,
        scratch_types=[
            pltpu.VMEM((hist, BBLK), jnp.int32),
            pltpu.VMEM((BBLK, d), table.dtype),
            pltpu.VMEM((BBLK, d), table.dtype),
            pltpu.VMEM((ct, TPC, 8, 128), table.dtype),
            pltpu.VMEM((ct, TPC, 8, 128), table.dtype),
            pltpu.SemaphoreType.DMA,
            pltpu.SemaphoreType.DMA,
            pltpu.SemaphoreType.DMA,
            pltpu.SemaphoreType.DMA,
        ],
        compiler_params=pltpu.CompilerParams(
            use_tc_tiling_on_sc=False, needs_layout_passes=False),
    )(table, idx_t)
    l5 = ltile.reshape(hist, ct, b // 128, 8, 128)
    return jnp.transpose(l5, (2, 4, 0, 1, 3)).reshape(b, hist, d)


def kernel(indices, table):
    b, h = indices.shape
    v, d = table.shape
    idx_t = jnp.transpose(indices).astype(jnp.int32)
    nfull = v // 128
    rem = v - nfull * 128
    if rem:
        tail = table[nfull * 128:].reshape(rem * d // 128, 128)
    else:
        tail = jnp.zeros((1, 128), table.dtype)
    table_lin = _sc_detile(jnp.transpose(table), tail, v, d).reshape(v, d)
    return _sc_emb(table_lin, idx_t, b, h, d)
